# baseline (speedup 1.0000x reference)
import functools

import jax
import jax.numpy as jnp
from jax import lax
from jax.experimental import pallas as pl
from jax.experimental.pallas import tpu as pltpu
from jax.experimental.pallas import tpu_sc as plsc

B, L, DIM = 1024, 200, 128
DAY_SIZE, TIME_SIZE = 32, 64
NUM_DAYS, DAILY_TIMES = 7, 288
DT = DAY_SIZE + TIME_SIZE
OUT_D = DIM + DT
N = B * L

_info = plsc.get_sparse_core_info()
NC, NS, LANES = _info.num_cores, _info.num_subcores, _info.num_lanes
NW = NC * NS
BATCH_PER_W = B // NW
CHUNK = 40
CPB = L // CHUNK
NCHUNK = BATCH_PER_W * CPB
NBUF = 4
NOUTER = NCHUNK // NBUF

_mesh = plsc.VectorSubcoreMesh(core_axis_name="c", subcore_axis_name="s")


@functools.partial(
    pl.kernel,
    out_type=jax.ShapeDtypeStruct((B, L, OUT_D), jnp.float32),
    mesh=_mesh,
    compiler_params=pltpu.CompilerParams(use_tc_tiling_on_sc=True),
    scratch_types=(
        [pltpu.VMEM((2 * CHUNK + LANES,), jnp.int32)] * NBUF
        + [pltpu.VMEM((CHUNK, DIM), jnp.float32)] * NBUF
        + [pltpu.VMEM((CHUNK, DT), jnp.float32)] * NBUF
        + [pltpu.VMEM((NUM_DAYS * DAY_SIZE,), jnp.float32)]
        + [pltpu.VMEM((DAILY_TIMES * TIME_SIZE,), jnp.float32)]
        + [pltpu.SemaphoreType.DMA] * (2 * NBUF)
    ),
)
def _sc_body(inp_hbm, idx_hbm, day_hbm, time_hbm, out_hbm, *scratch):
    idxraw_v = scratch[0:NBUF]
    inp_v = scratch[NBUF:2 * NBUF]
    dt_v = scratch[2 * NBUF:3 * NBUF]
    day_tab = scratch[3 * NBUF]
    time_tab = scratch[3 * NBUF + 1]
    in_sem = scratch[3 * NBUF + 2:3 * NBUF + 2 + NBUF]
    out_sem = scratch[3 * NBUF + 2 + NBUF:3 * NBUF + 2 + 2 * NBUF]

    wid = lax.axis_index("s") * NC + lax.axis_index("c")
    base_b = wid * BATCH_PER_W

    def chunk_pos(g):
        bb = base_b + g // CPB
        l0 = (g % CPB) * CHUNK
        return bb, l0

    def fire_in(g, b):
        bb, l0 = chunk_pos(g)
        r0 = bb * L + l0
        pltpu.async_copy(idx_hbm.at[pl.ds(2 * r0, 2 * CHUNK)],
                         idxraw_v[b].at[pl.ds(0, 2 * CHUNK)], in_sem[b])
        pltpu.async_copy(inp_hbm.at[bb, pl.ds(l0, CHUNK)], inp_v[b],
                         in_sem[b])

    def wait_in(b):
        pltpu.make_async_copy(idx_hbm.at[pl.ds(0, 2 * CHUNK)],
                              idxraw_v[b].at[pl.ds(0, 2 * CHUNK)],
                              in_sem[b]).wait()
        pltpu.make_async_copy(inp_hbm.at[0, pl.ds(0, CHUNK)],
                              inp_v[b], in_sem[b]).wait()

    def fire_out(g, b):
        bb, l0 = chunk_pos(g)
        pltpu.async_copy(
            inp_v[b], out_hbm.at[bb, pl.ds(l0, CHUNK), pl.ds(0, DIM)],
            out_sem[b])
        pltpu.async_copy(
            dt_v[b], out_hbm.at[bb, pl.ds(l0, CHUNK), pl.ds(DIM, DT)],
            out_sem[b])

    def wait_out(b):
        pltpu.make_async_copy(
            inp_v[b], out_hbm.at[0, pl.ds(0, CHUNK), pl.ds(0, DIM)],
            out_sem[b]).wait()
        pltpu.make_async_copy(
            dt_v[b], out_hbm.at[0, pl.ds(0, CHUNK), pl.ds(DIM, DT)],
            out_sem[b]).wait()

    pltpu.sync_copy(day_hbm, day_tab)
    pltpu.sync_copy(time_hbm, time_tab)

    for g0 in range(NBUF - 1):
        fire_in(g0, g0)

    @pl.loop(0, NOUTER)
    def _blk(k):
        for j in range(NBUF):
            g = k * NBUF + j
            b = j

            wait_in(b)

            f = g + NBUF - 1
            fb = (j + NBUF - 1) % NBUF

            @pl.when(f < NCHUNK)
            def _():
                @pl.when(g >= 1)
                def _():
                    wait_out(fb)
                fire_in(f, fb)

            @pl.loop(0, CHUNK, unroll=8)
            def _row(r):
                pair = idxraw_v[b][pl.ds(2 * r, LANES)]
                do = DAY_SIZE * pair[0]
                to = TIME_SIZE * pair[1]
                for c in range(0, DAY_SIZE, LANES):
                    dt_v[b][r, pl.ds(c, LANES)] = day_tab[pl.ds(do + c,
                                                                LANES)]
                for c in range(0, TIME_SIZE, LANES):
                    dt_v[b][r, pl.ds(DAY_SIZE + c, LANES)] = time_tab[
                        pl.ds(to + c, LANES)]

            fire_out(g, b)

    for g in range(NCHUNK - NBUF, NCHUNK):
        wait_out(g % NBUF)


def kernel(inp, daytime, emb_day, emb_time):
    idx = daytime.astype(jnp.int32).reshape(2 * N)
    return _sc_body(inp, idx,
                    emb_day.reshape(NUM_DAYS * DAY_SIZE),
                    emb_time.reshape(DAILY_TIMES * TIME_SIZE))
